# Initial kernel scaffold; baseline (speedup 1.0000x reference)
import jax
import jax.numpy as jnp
from jax.experimental import pallas as pl


def kernel(memories, queries, memory_lengths, query_lengths, T_query, T_in, T_out, W):
    raise NotImplementedError("write your pallas kernel here")


# speedup vs baseline: 1.0307x; 1.0307x over previous
import functools

import jax
import jax.numpy as jnp
from jax import lax
from jax.experimental import pallas as pl
from jax.experimental.pallas import tpu as pltpu
from jax.experimental.pallas import tpu_sc as plsc

_B = 1024
_M = 50
_D = 64
_LMEM = 7
_LQ = 19
_HOPS = 2

_NC = 2
_NS = 16
_NW = _NC * _NS

_N1 = _B * _M
_ROWS_W = _N1 // _NW
_CH = 80
_NCH = _ROWS_W // _CH
_QROWS = _B // _NW


def _sc_body(tin, tout, tq, idxm, idxq, sum_in, sum_out, sum_q,
             idx_v, acc_in, acc_out, idxq_v, acc_q, sem_a, sem_b):
    wid = lax.axis_index("s") * _NC + lax.axis_index("c")
    row0 = wid * _ROWS_W

    def chunk(ch, carry):
        base = pl.multiple_of(row0 + ch * _CH, 8)
        pltpu.sync_copy(idxm.at[wid * _NCH + ch], idx_v)
        for l in range(_LMEM):
            a = pltpu.async_copy(tin.at[idx_v.at[l]], acc_in, sem_a, add=(l > 0))
            b = pltpu.async_copy(tout.at[idx_v.at[l]], acc_out, sem_b, add=(l > 0))
            a.wait()
            b.wait()
        pltpu.sync_copy(acc_in, sum_in.at[pl.ds(base, _CH)])
        pltpu.sync_copy(acc_out, sum_out.at[pl.ds(base, _CH)])
        return carry

    lax.fori_loop(0, _NCH, chunk, 0)

    qbase = pl.multiple_of(wid * _QROWS, 8)
    pltpu.sync_copy(idxq.at[wid], idxq_v)
    for l in range(_LQ):
        pltpu.async_copy(tq.at[idxq_v.at[l]], acc_q, sem_a, add=(l > 0)).wait()
    pltpu.sync_copy(acc_q, sum_q.at[pl.ds(qbase, _QROWS)])


@functools.partial(jax.jit, static_argnums=())
def _sc_pool(T_in, T_out, T_query, idx_mem, idx_q):
    f = pl.kernel(
        _sc_body,
        out_type=(
            jax.ShapeDtypeStruct((_N1, _D), jnp.float32),
            jax.ShapeDtypeStruct((_N1, _D), jnp.float32),
            jax.ShapeDtypeStruct((_B, _D), jnp.float32),
        ),
        mesh=plsc.VectorSubcoreMesh(core_axis_name="c", subcore_axis_name="s"),
        scratch_types=[
            pltpu.VMEM((_LMEM, _CH), jnp.int32),
            pltpu.VMEM((_CH, _D), jnp.float32),
            pltpu.VMEM((_CH, _D), jnp.float32),
            pltpu.VMEM((_LQ, _QROWS), jnp.int32),
            pltpu.VMEM((_QROWS, _D), jnp.float32),
            pltpu.SemaphoreType.DMA,
            pltpu.SemaphoreType.DMA,
        ],
        compiler_params=pltpu.CompilerParams(use_tc_tiling_on_sc=False),
    )
    return f(T_in, T_out, T_query, idx_mem, idx_q)


def _hops_body(ml_ref, ql_ref, w_ref, sin_ref, sout_ref, sq_ref, out_ref):
    f32 = jnp.float32
    ml = ml_ref[...]
    inv_m = 1.0 / jnp.maximum(ml, 1).astype(f32)
    in_mem = sin_ref[...] * inv_m[:, :, None]
    out_mem = sout_ref[...] * inv_m[:, :, None]
    q = sq_ref[...] * (1.0 / jnp.maximum(ql_ref[...], 1).astype(f32))
    w = w_ref[...]
    valid = ml != 0
    neg = jnp.float32(-1e20)
    for _ in range(_HOPS):
        att = jnp.sum(in_mem * q[:, None, :], axis=2)
        att = jnp.where(valid, att, neg)
        att = att - jnp.max(att, axis=1, keepdims=True)
        p = jnp.exp(att)
        p = p / jnp.sum(p, axis=1, keepdims=True)
        mem_out = jnp.sum(p[:, :, None] * out_mem, axis=1)
        q = mem_out + lax.dot_general(q, w, (((1,), (1,)), ((), ())),
                                      preferred_element_type=f32)
    out_ref[...] = q


def _hops(memory_lengths, query_lengths2, W, sum_in, sum_out, sum_q):
    bb = 128
    return pl.pallas_call(
        _hops_body,
        grid=(_B // bb,),
        in_specs=[
            pl.BlockSpec((bb, _M), lambda i: (i, 0)),
            pl.BlockSpec((bb, 1), lambda i: (i, 0)),
            pl.BlockSpec((_D, _D), lambda i: (0, 0)),
            pl.BlockSpec((bb, _M, _D), lambda i: (i, 0, 0)),
            pl.BlockSpec((bb, _M, _D), lambda i: (i, 0, 0)),
            pl.BlockSpec((bb, _D), lambda i: (i, 0)),
        ],
        out_specs=pl.BlockSpec((bb, _D), lambda i: (i, 0)),
        out_shape=jax.ShapeDtypeStruct((_B, _D), jnp.float32),
    )(memory_lengths, query_lengths2, W, sum_in, sum_out, sum_q)


def kernel(memories, queries, memory_lengths, query_lengths, T_query, T_in, T_out, W):
    memories = memories.astype(jnp.int32)
    queries = queries.astype(jnp.int32)

    fl = memory_lengths.reshape(-1).astype(jnp.int32)
    off = (jnp.cumsum(fl) - fl).astype(jnp.int32)
    lr = jnp.arange(_LMEM, dtype=jnp.int32)
    pos = off[None, :] + lr[:, None]
    ok = lr[:, None] < fl[None, :]
    idx_mem = jnp.where(ok, memories[jnp.clip(pos, 0, memories.shape[0] - 1)], 0)
    idx_mem = idx_mem.reshape(_LMEM, _NW * _NCH, _CH).transpose(1, 0, 2)

    qfl = query_lengths.astype(jnp.int32)
    qoff = (jnp.cumsum(qfl) - qfl).astype(jnp.int32)
    qlr = jnp.arange(_LQ, dtype=jnp.int32)
    qpos = qoff[None, :] + qlr[:, None]
    qok = qlr[:, None] < qfl[None, :]
    idx_q = jnp.where(qok, queries[jnp.clip(qpos, 0, queries.shape[0] - 1)], 0)
    idx_q = idx_q.reshape(_LQ, _NW, _QROWS).transpose(1, 0, 2)

    sum_in, sum_out, sum_q = _sc_pool(T_in, T_out, T_query, idx_mem, idx_q)

    return _hops(
        memory_lengths.astype(jnp.int32),
        qfl[:, None],
        W,
        sum_in.reshape(_B, _M, _D),
        sum_out.reshape(_B, _M, _D),
        sum_q,
    )


# speedup vs baseline: 1.0359x; 1.0051x over previous
import functools

import jax
import jax.numpy as jnp
from jax import lax
from jax.experimental import pallas as pl
from jax.experimental.pallas import tpu as pltpu
from jax.experimental.pallas import tpu_sc as plsc

_B = 1024
_M = 50
_D = 64
_LMEM = 7
_LQ = 19
_HOPS = 2

_NC = 2
_NS = 16
_NW = _NC * _NS

_N1 = _B * _M
_ROWS_W = _N1 // _NW
_CH = 80
_NCH = _ROWS_W // _CH
_QROWS = _B // _NW


_NSLOT = 4
_NGRP = _NCH // _NSLOT
_ZV = 4


def _zero_acc(ref, slot):
    z = jnp.zeros((16,), jnp.float32)

    def row(r, carry):
        for c in range(_ZV):
            ref[slot, r, pl.ds(c * 16, 16)] = z
        return carry

    lax.fori_loop(0, _CH, row, 0)


def _sc_body(tin, tout, tq, idxm, idxq, sum_in, sum_out, sum_q,
             idx_v, acc_in, acc_out, idxq_v, acc_q, sem_g, sem_i, sem_s):
    wid = lax.axis_index("s") * _NC + lax.axis_index("c")
    row0 = wid * _ROWS_W

    for s in range(_NSLOT):
        _zero_acc(acc_in, s)
        _zero_acc(acc_out, s)

    def group(g, carry):
        idx_cps = [
            pltpu.async_copy(idxm.at[wid * _NCH + g * _NSLOT + s], idx_v.at[s], sem_i)
            for s in range(_NSLOT)
        ]
        for cp in idx_cps:
            cp.wait()
        cps = []
        for s in range(_NSLOT):
            for l in range(_LMEM):
                cps.append(pltpu.async_copy(
                    tin.at[idx_v.at[s, l]], acc_in.at[s], sem_g, add=True))
                cps.append(pltpu.async_copy(
                    tout.at[idx_v.at[s, l]], acc_out.at[s], sem_g, add=True))
        for cp in cps:
            cp.wait()
        st = []
        for s in range(_NSLOT):
            base = pl.multiple_of(row0 + (g * _NSLOT + s) * _CH, 8)
            st.append(pltpu.async_copy(acc_in.at[s], sum_in.at[pl.ds(base, _CH)], sem_s))
            st.append(pltpu.async_copy(acc_out.at[s], sum_out.at[pl.ds(base, _CH)], sem_s))
        for cp in st:
            cp.wait()
        for s in range(_NSLOT):
            _zero_acc(acc_in, s)
            _zero_acc(acc_out, s)
        return carry

    lax.fori_loop(0, _NGRP, group, 0)

    def qrow(r, carry):
        for c in range(_ZV):
            acc_q[r, pl.ds(c * 16, 16)] = jnp.zeros((16,), jnp.float32)
        return carry

    lax.fori_loop(0, _QROWS, qrow, 0)
    qbase = pl.multiple_of(wid * _QROWS, 8)
    pltpu.sync_copy(idxq.at[wid], idxq_v)
    qcps = [pltpu.async_copy(tq.at[idxq_v.at[l]], acc_q, sem_g, add=True)
            for l in range(_LQ)]
    for cp in qcps:
        cp.wait()
    pltpu.sync_copy(acc_q, sum_q.at[pl.ds(qbase, _QROWS)])


@functools.partial(jax.jit, static_argnums=())
def _sc_pool(T_in, T_out, T_query, idx_mem, idx_q):
    f = pl.kernel(
        _sc_body,
        out_type=(
            jax.ShapeDtypeStruct((_N1, _D), jnp.float32),
            jax.ShapeDtypeStruct((_N1, _D), jnp.float32),
            jax.ShapeDtypeStruct((_B, _D), jnp.float32),
        ),
        mesh=plsc.VectorSubcoreMesh(core_axis_name="c", subcore_axis_name="s"),
        scratch_types=[
            pltpu.VMEM((_NSLOT, _LMEM, _CH), jnp.int32),
            pltpu.VMEM((_NSLOT, _CH, _D), jnp.float32),
            pltpu.VMEM((_NSLOT, _CH, _D), jnp.float32),
            pltpu.VMEM((_LQ, _QROWS), jnp.int32),
            pltpu.VMEM((_QROWS, _D), jnp.float32),
            pltpu.SemaphoreType.DMA,
            pltpu.SemaphoreType.DMA,
            pltpu.SemaphoreType.DMA,
        ],
        compiler_params=pltpu.CompilerParams(use_tc_tiling_on_sc=False),
    )
    return f(T_in, T_out, T_query, idx_mem, idx_q)


def _hops_body(ml_ref, ql_ref, w_ref, sin_ref, sout_ref, sq_ref, out_ref):
    f32 = jnp.float32
    ml = ml_ref[...]
    inv_m = 1.0 / jnp.maximum(ml, 1).astype(f32)
    in_mem = sin_ref[...] * inv_m[:, :, None]
    out_mem = sout_ref[...] * inv_m[:, :, None]
    q = sq_ref[...] * (1.0 / jnp.maximum(ql_ref[...], 1).astype(f32))
    w = w_ref[...]
    valid = ml != 0
    neg = jnp.float32(-1e20)
    for _ in range(_HOPS):
        att = jnp.sum(in_mem * q[:, None, :], axis=2)
        att = jnp.where(valid, att, neg)
        att = att - jnp.max(att, axis=1, keepdims=True)
        p = jnp.exp(att)
        p = p / jnp.sum(p, axis=1, keepdims=True)
        mem_out = jnp.sum(p[:, :, None] * out_mem, axis=1)
        q = mem_out + lax.dot_general(q, w, (((1,), (1,)), ((), ())),
                                      preferred_element_type=f32)
    out_ref[...] = q


def _hops(memory_lengths, query_lengths2, W, sum_in, sum_out, sum_q):
    bb = 128
    return pl.pallas_call(
        _hops_body,
        grid=(_B // bb,),
        in_specs=[
            pl.BlockSpec((bb, _M), lambda i: (i, 0)),
            pl.BlockSpec((bb, 1), lambda i: (i, 0)),
            pl.BlockSpec((_D, _D), lambda i: (0, 0)),
            pl.BlockSpec((bb, _M, _D), lambda i: (i, 0, 0)),
            pl.BlockSpec((bb, _M, _D), lambda i: (i, 0, 0)),
            pl.BlockSpec((bb, _D), lambda i: (i, 0)),
        ],
        out_specs=pl.BlockSpec((bb, _D), lambda i: (i, 0)),
        out_shape=jax.ShapeDtypeStruct((_B, _D), jnp.float32),
    )(memory_lengths, query_lengths2, W, sum_in, sum_out, sum_q)


def kernel(memories, queries, memory_lengths, query_lengths, T_query, T_in, T_out, W):
    memories = memories.astype(jnp.int32)
    queries = queries.astype(jnp.int32)

    fl = memory_lengths.reshape(-1).astype(jnp.int32)
    off = (jnp.cumsum(fl) - fl).astype(jnp.int32)
    lr = jnp.arange(_LMEM, dtype=jnp.int32)
    pos = off[None, :] + lr[:, None]
    ok = lr[:, None] < fl[None, :]
    idx_mem = jnp.where(ok, memories[jnp.clip(pos, 0, memories.shape[0] - 1)], 0)
    idx_mem = idx_mem.reshape(_LMEM, _NW * _NCH, _CH).transpose(1, 0, 2)

    qfl = query_lengths.astype(jnp.int32)
    qoff = (jnp.cumsum(qfl) - qfl).astype(jnp.int32)
    qlr = jnp.arange(_LQ, dtype=jnp.int32)
    qpos = qoff[None, :] + qlr[:, None]
    qok = qlr[:, None] < qfl[None, :]
    idx_q = jnp.where(qok, queries[jnp.clip(qpos, 0, queries.shape[0] - 1)], 0)
    idx_q = idx_q.reshape(_LQ, _NW, _QROWS).transpose(1, 0, 2)

    sum_in, sum_out, sum_q = _sc_pool(T_in, T_out, T_query, idx_mem, idx_q)

    return _hops(
        memory_lengths.astype(jnp.int32),
        qfl[:, None],
        W,
        sum_in.reshape(_B, _M, _D),
        sum_out.reshape(_B, _M, _D),
        sum_q,
    )


# speedup vs baseline: 8.9855x; 8.6738x over previous
import functools

import jax
import jax.numpy as jnp
from jax import lax
from jax.experimental import pallas as pl
from jax.experimental.pallas import tpu as pltpu
from jax.experimental.pallas import tpu_sc as plsc

_B = 1024
_M = 50
_D = 64
_LMEM = 7
_LQ = 19
_HOPS = 2

_NC = 2
_NS = 16
_NW = _NC * _NS

_N1 = _B * _M
_ROWS_W = _N1 // _NW
_CH = 80
_NCH = _ROWS_W // _CH
_TOK = _CH * _LMEM
_SUB = 40
_NSUB = _TOK // _SUB
_QROWS = _B // _NW
_QTOK = _QROWS * _LQ
_QNSUB = (_QTOK + _SUB - 1) // _SUB
_RBUF = _QNSUB * _SUB + 8


def _sc_body(tin, tout, tq, mem_pad, off_ext, q_pad, qoff_ext,
             sum_in, sum_out, sum_q,
             off_v, tok_v, rows_in, rows_out, res_in, res_out, res_q,
             sem_a, sem_b, sem_t):
    wid = lax.axis_index("s") * _NC + lax.axis_index("c")
    row0 = wid * _ROWS_W

    def chunk(ch, carry):
        base = pl.multiple_of(row0 + ch * _CH, 8)
        pltpu.sync_copy(off_ext.at[pl.ds(base, _CH + 8)], off_v.at[pl.ds(0, _CH + 8)])
        t0 = off_v[pl.ds(0, 16)][0]
        base_tok = pl.multiple_of((t0 // 8) * 8, 8)
        pltpu.sync_copy(mem_pad.at[pl.ds(base_tok, _RBUF)], tok_v)
        nrows = off_v[pl.ds(_CH - 8, 16)][8] - base_tok
        cps = [None] * (_NSUB + 1)
        for j in range(_NSUB + 1):
            @pl.when(j * _SUB < nrows)
            def _(j=j):
                cps[j] = (
                    pltpu.async_copy(
                        tin.at[tok_v.at[pl.ds(j * _SUB, _SUB)]],
                        rows_in.at[pl.ds(j * _SUB, _SUB)], sem_a),
                    pltpu.async_copy(
                        tout.at[tok_v.at[pl.ds(j * _SUB, _SUB)]],
                        rows_out.at[pl.ds(j * _SUB, _SUB)], sem_b),
                )
        for j in range(_NSUB + 1):
            @pl.when(j * _SUB < nrows)
            def _(j=j):
                cps[j][0].wait()
                cps[j][1].wait()

        def slot8(g, c2):
            va = off_v[pl.ds(g * 8, 16)]
            for i in range(8):
                s = va[i] - base_tok
                ln = va[i + 1] - va[i]
                r = g * 8 + i
                for c in range(_D // 16):
                    acc_i = jnp.zeros((16,), jnp.float32)
                    acc_o = jnp.zeros((16,), jnp.float32)
                    for jj in range(_LMEM):
                        keep = jj < ln
                        zi = jnp.zeros((16,), jnp.float32)
                        acc_i = acc_i + jnp.where(keep, rows_in[s + jj, pl.ds(c * 16, 16)], zi)
                        acc_o = acc_o + jnp.where(keep, rows_out[s + jj, pl.ds(c * 16, 16)], zi)
                    res_in[r, pl.ds(c * 16, 16)] = acc_i
                    res_out[r, pl.ds(c * 16, 16)] = acc_o
            return c2

        lax.fori_loop(0, _CH // 8, slot8, 0)
        pltpu.sync_copy(res_in, sum_in.at[pl.ds(base, _CH)])
        pltpu.sync_copy(res_out, sum_out.at[pl.ds(base, _CH)])
        return carry

    lax.fori_loop(0, _NCH, chunk, 0)

    qbase = pl.multiple_of(wid * _QROWS, 8)
    pltpu.sync_copy(qoff_ext.at[pl.ds(qbase, _QROWS + 8)], off_v.at[pl.ds(0, _QROWS + 8)])
    qt0 = off_v[pl.ds(0, 16)][0]
    qbase_tok = pl.multiple_of((qt0 // 8) * 8, 8)
    pltpu.sync_copy(q_pad.at[pl.ds(qbase_tok, _RBUF)], tok_v)
    qnrows = off_v[pl.ds(_QROWS - 8, 16)][8] - qbase_tok
    qcps = [None] * _QNSUB
    for j in range(_QNSUB):
        @pl.when(j * _SUB < qnrows)
        def _(j=j):
            qcps[j] = pltpu.async_copy(
                tq.at[tok_v.at[pl.ds(j * _SUB, _SUB)]],
                rows_in.at[pl.ds(j * _SUB, _SUB)], sem_a)
    for j in range(_QNSUB):
        @pl.when(j * _SUB < qnrows)
        def _(j=j):
            qcps[j].wait()

    def qslot8(g, c2):
        va = off_v[pl.ds(g * 8, 16)]
        for i in range(8):
            s = va[i] - qbase_tok
            ln = va[i + 1] - va[i]
            r = g * 8 + i
            for c in range(_D // 16):
                acc = jnp.zeros((16,), jnp.float32)
                for jj in range(_LQ):
                    acc = acc + jnp.where(jj < ln, rows_in[s + jj, pl.ds(c * 16, 16)],
                                          jnp.zeros((16,), jnp.float32))
                res_q[r, pl.ds(c * 16, 16)] = acc
        return c2

    lax.fori_loop(0, _QROWS // 8, qslot8, 0)
    pltpu.sync_copy(res_q, sum_q.at[pl.ds(qbase, _QROWS)])


def _sc_pool(T_in, T_out, T_query, mem_pad, off_ext, q_pad, qoff_ext):
    f = pl.kernel(
        _sc_body,
        out_type=(
            jax.ShapeDtypeStruct((_N1, _D), jnp.float32),
            jax.ShapeDtypeStruct((_N1, _D), jnp.float32),
            jax.ShapeDtypeStruct((_B, _D), jnp.float32),
        ),
        mesh=plsc.VectorSubcoreMesh(core_axis_name="c", subcore_axis_name="s"),
        scratch_types=[
            pltpu.VMEM((_CH + 24,), jnp.int32),
            pltpu.VMEM((_RBUF,), jnp.int32),
            pltpu.VMEM((_RBUF, _D), jnp.float32),
            pltpu.VMEM(((_NSUB + 1) * _SUB + 8, _D), jnp.float32),
            pltpu.VMEM((_CH, _D), jnp.float32),
            pltpu.VMEM((_CH, _D), jnp.float32),
            pltpu.VMEM((_QROWS, _D), jnp.float32),
            pltpu.SemaphoreType.DMA,
            pltpu.SemaphoreType.DMA,
            pltpu.SemaphoreType.DMA,
        ],
        compiler_params=pltpu.CompilerParams(use_tc_tiling_on_sc=False),
    )
    return f(T_in, T_out, T_query, mem_pad, off_ext, q_pad, qoff_ext)


def _hops_body(ml_ref, ql_ref, w_ref, sin_ref, sout_ref, sq_ref, out_ref):
    f32 = jnp.float32
    ml = ml_ref[...]
    inv_m = 1.0 / jnp.maximum(ml, 1).astype(f32)
    in_mem = sin_ref[...] * inv_m[:, :, None]
    out_mem = sout_ref[...] * inv_m[:, :, None]
    q = sq_ref[...] * (1.0 / jnp.maximum(ql_ref[...], 1).astype(f32))
    w = w_ref[...]
    valid = ml != 0
    neg = jnp.float32(-1e20)
    for _ in range(_HOPS):
        att = jnp.sum(in_mem * q[:, None, :], axis=2)
        att = jnp.where(valid, att, neg)
        att = att - jnp.max(att, axis=1, keepdims=True)
        p = jnp.exp(att)
        p = p / jnp.sum(p, axis=1, keepdims=True)
        mem_out = jnp.sum(p[:, :, None] * out_mem, axis=1)
        q = mem_out + lax.dot_general(q, w, (((1,), (1,)), ((), ())),
                                      preferred_element_type=f32)
    out_ref[...] = q


def _hops(memory_lengths, query_lengths2, W, sum_in, sum_out, sum_q):
    bb = 128
    return pl.pallas_call(
        _hops_body,
        grid=(_B // bb,),
        in_specs=[
            pl.BlockSpec((bb, _M), lambda i: (i, 0)),
            pl.BlockSpec((bb, 1), lambda i: (i, 0)),
            pl.BlockSpec((_D, _D), lambda i: (0, 0)),
            pl.BlockSpec((bb, _M, _D), lambda i: (i, 0, 0)),
            pl.BlockSpec((bb, _M, _D), lambda i: (i, 0, 0)),
            pl.BlockSpec((bb, _D), lambda i: (i, 0)),
        ],
        out_specs=pl.BlockSpec((bb, _D), lambda i: (i, 0)),
        out_shape=jax.ShapeDtypeStruct((_B, _D), jnp.float32),
    )(memory_lengths, query_lengths2, W, sum_in, sum_out, sum_q)


def kernel(memories, queries, memory_lengths, query_lengths, T_query, T_in, T_out, W):
    memories = memories.astype(jnp.int32)
    queries = queries.astype(jnp.int32)

    fl = memory_lengths.reshape(-1).astype(jnp.int32)
    csum = jnp.cumsum(fl)
    off_ext = jnp.concatenate([jnp.zeros((1,), jnp.int32), csum,
                               jnp.full((8,), csum[-1], jnp.int32)])
    mem_pad = jnp.concatenate([memories, jnp.zeros((_RBUF,), jnp.int32)])

    qfl = query_lengths.astype(jnp.int32)
    qcsum = jnp.cumsum(qfl)
    qoff_ext = jnp.concatenate([jnp.zeros((1,), jnp.int32), qcsum,
                                jnp.full((8,), qcsum[-1], jnp.int32)])
    q_pad = jnp.concatenate([queries, jnp.zeros((_RBUF,), jnp.int32)])

    sum_in, sum_out, sum_q = _sc_pool(T_in, T_out, T_query,
                                      mem_pad, off_ext, q_pad, qoff_ext)

    return _hops(
        memory_lengths.astype(jnp.int32),
        qfl[:, None],
        W,
        sum_in.reshape(_B, _M, _D),
        sum_out.reshape(_B, _M, _D),
        sum_q,
    )
